# Initial kernel scaffold; baseline (speedup 1.0000x reference)
import jax
import jax.numpy as jnp
from jax.experimental import pallas as pl


def kernel(x, batch, W_g1, b_g1, W_g2, b_g2, W_p, b_p, gamma, beta):
    raise NotImplementedError("write your pallas kernel here")


# speedup vs baseline: 3.2048x; 3.2048x over previous
import jax
import jax.numpy as jnp
from jax.experimental import pallas as pl

N = 100000
H = 128
G = 512
B = 1000
NB = N // B
NEG = -3.0e38


def _gate(x, wg1, bg1, wg2):
    h = jnp.maximum(
        jnp.dot(x, wg1, preferred_element_type=jnp.float32) + bg1, 0.0
    )
    return jnp.dot(h, wg2, preferred_element_type=jnp.float32)


def _onehot(seg2d):
    iota = jax.lax.broadcasted_iota(jnp.int32, (B, G), 1)
    return (seg2d == iota).astype(jnp.float32)


def _pass1(x_ref, seg_ref, wg1_ref, bg1_ref, wg2_ref,
           sums_ref, counts_ref, maxs_ref, gmax_ref):
    i = pl.program_id(0)
    x = x_ref[...]
    seg = seg_ref[...]
    gate = _gate(x, wg1_ref[...], bg1_ref[...], wg2_ref[...])
    onehot = _onehot(seg)

    s_blk = jax.lax.dot_general(onehot, x, (((0,), (0,)), ((), ())),
                                preferred_element_type=jnp.float32)
    c_blk = jnp.sum(onehot, axis=0, keepdims=True)

    m = x
    gm = gate
    d = 1
    while d < B:
        seg_sh = jnp.concatenate(
            [jnp.full((d, 1), -1, jnp.int32), seg[:-d, :]], axis=0)
        ok = seg_sh == seg
        m_sh = jnp.concatenate(
            [jnp.full((d, H), NEG, jnp.float32), m[:-d, :]], axis=0)
        m = jnp.maximum(m, jnp.where(ok, m_sh, NEG))
        g_sh = jnp.concatenate(
            [jnp.full((d, 1), NEG, jnp.float32), gm[:-d, :]], axis=0)
        gm = jnp.maximum(gm, jnp.where(ok, g_sh, NEG))
        d *= 2

    seg_nxt = jnp.concatenate(
        [seg[1:, :], jnp.full((1, 1), -1, jnp.int32)], axis=0)
    tail = (seg != seg_nxt).astype(jnp.float32)
    oh_tail = onehot * tail
    ind = jnp.sum(oh_tail, axis=0, keepdims=True)
    ind_col = ind.reshape(G, 1)
    mx_blk = jax.lax.dot_general(oh_tail, m, (((0,), (0,)), ((), ())),
                                 preferred_element_type=jnp.float32)
    gmx_blk = jax.lax.dot_general(oh_tail, gm, (((0,), (0,)), ((), ())),
                                  preferred_element_type=jnp.float32)
    neginf = float('-inf')
    mx_blk = jnp.where(ind_col > 0, mx_blk, neginf)
    gmx_blk = jnp.where(ind_col > 0, gmx_blk, neginf)

    @pl.when(i == 0)
    def _():
        sums_ref[...] = s_blk
        counts_ref[...] = c_blk
        maxs_ref[...] = mx_blk
        gmax_ref[...] = gmx_blk

    @pl.when(i > 0)
    def _():
        sums_ref[...] += s_blk
        counts_ref[...] += c_blk
        maxs_ref[...] = jnp.maximum(maxs_ref[...], mx_blk)
        gmax_ref[...] = jnp.maximum(gmax_ref[...], gmx_blk)


def _pass2(x_ref, seg_ref, wg1_ref, bg1_ref, wg2_ref, gmax_ref,
           esum_ref, exsum_ref):
    i = pl.program_id(0)
    x = x_ref[...]
    seg = seg_ref[...]
    gate = _gate(x, wg1_ref[...], bg1_ref[...], wg2_ref[...])
    onehot = _onehot(seg)
    gmaxc = jnp.maximum(gmax_ref[...], NEG)
    gathered = jnp.dot(onehot, gmaxc,
                       preferred_element_type=jnp.float32)
    e = jnp.exp(gate - gathered)
    es_blk = jax.lax.dot_general(onehot, e, (((0,), (0,)), ((), ())),
                                 preferred_element_type=jnp.float32)
    y = x * e
    ex_blk = jax.lax.dot_general(onehot, y, (((0,), (0,)), ((), ())),
                                 preferred_element_type=jnp.float32)

    @pl.when(i == 0)
    def _():
        esum_ref[...] = es_blk
        exsum_ref[...] = ex_blk

    @pl.when(i > 0)
    def _():
        esum_ref[...] += es_blk
        exsum_ref[...] += ex_blk


def _pass3(sums_ref, counts_ref, maxs_ref, esum_ref, exsum_ref,
           wpa_ref, wpb_ref, wpc_ref, bp_ref, gamma_ref, beta_ref, out_ref):
    counts = counts_ref[...].reshape(G, 1)
    z_mean = sums_ref[...] / jnp.maximum(counts, 1.0)
    z_max = maxs_ref[...]
    z_attn = exsum_ref[...] / jnp.maximum(esum_ref[...], 1e-30)
    z = (jnp.dot(z_mean, wpa_ref[...], preferred_element_type=jnp.float32)
         + jnp.dot(z_max, wpb_ref[...], preferred_element_type=jnp.float32)
         + jnp.dot(z_attn, wpc_ref[...], preferred_element_type=jnp.float32)
         + bp_ref[...])
    mu = jnp.mean(z, axis=1, keepdims=True)
    var = jnp.mean((z - mu) ** 2, axis=1, keepdims=True)
    out_ref[...] = ((z - mu) * jax.lax.rsqrt(var + 1e-5) * gamma_ref[...]
                    + beta_ref[...])


def kernel(x, batch, W_g1, b_g1, W_g2, b_g2, W_p, b_p, gamma, beta):
    seg = batch.astype(jnp.int32).reshape(N, 1)
    bg1 = b_g1.reshape(1, H // 4)

    full = lambda shp: pl.BlockSpec(shp, lambda i: tuple(0 for _ in shp))
    sums, counts, maxs, gmax = pl.pallas_call(
        _pass1,
        grid=(NB,),
        in_specs=[
            pl.BlockSpec((B, H), lambda i: (i, 0)),
            pl.BlockSpec((B, 1), lambda i: (i, 0)),
            full((H, H // 4)),
            full((1, H // 4)),
            full((H // 4, 1)),
        ],
        out_specs=[full((G, H)), full((1, G)), full((G, H)), full((G, 1))],
        out_shape=[
            jax.ShapeDtypeStruct((G, H), jnp.float32),
            jax.ShapeDtypeStruct((1, G), jnp.float32),
            jax.ShapeDtypeStruct((G, H), jnp.float32),
            jax.ShapeDtypeStruct((G, 1), jnp.float32),
        ],
    )(x, seg, W_g1, bg1, W_g2)

    esum, exsum = pl.pallas_call(
        _pass2,
        grid=(NB,),
        in_specs=[
            pl.BlockSpec((B, H), lambda i: (i, 0)),
            pl.BlockSpec((B, 1), lambda i: (i, 0)),
            full((H, H // 4)),
            full((1, H // 4)),
            full((H // 4, 1)),
            full((G, 1)),
        ],
        out_specs=[full((G, 1)), full((G, H))],
        out_shape=[
            jax.ShapeDtypeStruct((G, 1), jnp.float32),
            jax.ShapeDtypeStruct((G, H), jnp.float32),
        ],
    )(x, seg, W_g1, bg1, W_g2, gmax)

    out = pl.pallas_call(
        _pass3,
        out_shape=jax.ShapeDtypeStruct((G, H), jnp.float32),
    )(sums, counts, maxs, esum, exsum,
      W_p[:H], W_p[H:2 * H], W_p[2 * H:], b_p.reshape(1, H),
      gamma.reshape(1, H), beta.reshape(1, H))
    return out


# speedup vs baseline: 3.6246x; 1.1310x over previous
import jax
import jax.numpy as jnp
from jax.experimental import pallas as pl
from jax.experimental.pallas import tpu as pltpu

N = 100000
H = 128
G = 512
B = 1000
NB = N // B
NEG = -3.0e38


def _fused(x_ref, seg_ref, wg1_ref, bg1_ref, wg2_ref,
           wpa_ref, wpb_ref, wpc_ref, bp_ref, gamma_ref, beta_ref,
           out_ref,
           sums, counts, maxs, rmax, esum, exsum):
    i = pl.program_id(0)
    x = x_ref[...]
    xb = x.astype(jnp.bfloat16)
    seg = seg_ref[...]

    h = jnp.maximum(
        jnp.dot(xb, wg1_ref[...], preferred_element_type=jnp.float32)
        + bg1_ref[...], 0.0)
    gate = jnp.dot(h.astype(jnp.bfloat16), wg2_ref[...],
                   preferred_element_type=jnp.float32)

    iota = jax.lax.broadcasted_iota(jnp.int32, (B, G), 1)
    onehot = (seg == iota).astype(jnp.bfloat16)

    s_blk = jax.lax.dot_general(onehot, xb, (((0,), (0,)), ((), ())),
                                preferred_element_type=jnp.float32)
    c_blk = jnp.sum(onehot.astype(jnp.float32), axis=0, keepdims=True)

    m = xb
    gm = gate
    d = 1
    while d < B:
        seg_sh = jnp.concatenate(
            [jnp.full((d, 1), -1, jnp.int32), seg[:-d, :]], axis=0)
        ok = seg_sh == seg
        m_sh = jnp.concatenate(
            [jnp.full((d, H), NEG, jnp.bfloat16), m[:-d, :]], axis=0)
        m = jnp.maximum(m, jnp.where(ok, m_sh, jnp.bfloat16(NEG)))
        g_sh = jnp.concatenate(
            [jnp.full((d, 1), NEG, jnp.float32), gm[:-d, :]], axis=0)
        gm = jnp.maximum(gm, jnp.where(ok, g_sh, NEG))
        d *= 2

    seg_nxt = jnp.concatenate(
        [seg[1:, :], jnp.full((1, 1), -1, jnp.int32)], axis=0)
    tail = seg != seg_nxt
    oh_tail = jnp.where(tail, onehot, jnp.bfloat16(0))
    ind_col = jax.lax.dot_general(
        oh_tail, jnp.ones((B, 1), jnp.bfloat16), (((0,), (0,)), ((), ())),
        preferred_element_type=jnp.float32)
    mx_blk = jax.lax.dot_general(oh_tail, m, (((0,), (0,)), ((), ())),
                                 preferred_element_type=jnp.float32)
    gmx_blk = jax.lax.dot_general(
        oh_tail, gm.astype(jnp.bfloat16), (((0,), (0,)), ((), ())),
        preferred_element_type=jnp.float32)
    present = ind_col > 0
    mx_blk = jnp.where(present, mx_blk, NEG)
    gmx_blk = jnp.where(present, gmx_blk, NEG)

    gathered = jnp.dot(onehot, gmx_blk.astype(jnp.bfloat16),
                       preferred_element_type=jnp.float32)
    e = jnp.exp(gate - gathered)
    es_blk = jax.lax.dot_general(
        onehot, e.astype(jnp.bfloat16), (((0,), (0,)), ((), ())),
        preferred_element_type=jnp.float32)
    y = xb * e.astype(jnp.bfloat16)
    ex_blk = jax.lax.dot_general(onehot, y, (((0,), (0,)), ((), ())),
                                 preferred_element_type=jnp.float32)

    @pl.when(i == 0)
    def _():
        sums[...] = s_blk
        counts[...] = c_blk
        maxs[...] = mx_blk
        rmax[...] = gmx_blk
        scale = jnp.exp(gmx_blk - gmx_blk)
        esum[...] = es_blk * scale
        exsum[...] = ex_blk * scale

    @pl.when(i > 0)
    def _():
        sums[...] += s_blk
        counts[...] += c_blk
        maxs[...] = jnp.maximum(maxs[...], mx_blk)
        r_old = rmax[...]
        r_new = jnp.maximum(r_old, gmx_blk)
        scale_old = jnp.exp(r_old - r_new)
        scale_blk = jnp.exp(gmx_blk - r_new)
        esum[...] = esum[...] * scale_old + es_blk * scale_blk
        exsum[...] = exsum[...] * scale_old + ex_blk * scale_blk
        rmax[...] = r_new

    @pl.when(i == NB - 1)
    def _():
        cnt = counts[...].reshape(G, 1)
        nonempty = cnt > 0
        z_mean = sums[...] / jnp.maximum(cnt, 1.0)
        z_max = jnp.where(nonempty, maxs[...], float('-inf'))
        z_attn = exsum[...] / jnp.maximum(esum[...], 1e-30)
        z = (jnp.dot(z_mean, wpa_ref[...], preferred_element_type=jnp.float32)
             + jnp.dot(z_max, wpb_ref[...], preferred_element_type=jnp.float32)
             + jnp.dot(z_attn, wpc_ref[...],
                       preferred_element_type=jnp.float32)
             + bp_ref[...])
        mu = jnp.mean(z, axis=1, keepdims=True)
        var = jnp.mean((z - mu) ** 2, axis=1, keepdims=True)
        out_ref[...] = ((z - mu) * jax.lax.rsqrt(var + 1e-5) * gamma_ref[...]
                        + beta_ref[...])


def kernel(x, batch, W_g1, b_g1, W_g2, b_g2, W_p, b_p, gamma, beta):
    seg = batch.astype(jnp.int32).reshape(N, 1)
    bg1 = b_g1.reshape(1, H // 4)

    full = lambda shp: pl.BlockSpec(shp, lambda i: tuple(0 for _ in shp))
    out = pl.pallas_call(
        _fused,
        grid=(NB,),
        in_specs=[
            pl.BlockSpec((B, H), lambda i: (i, 0)),
            pl.BlockSpec((B, 1), lambda i: (i, 0)),
            full((H, H // 4)),
            full((1, H // 4)),
            full((H // 4, 1)),
            full((H, H)), full((H, H)), full((H, H)),
            full((1, H)), full((1, H)), full((1, H)),
        ],
        out_specs=full((G, H)),
        out_shape=jax.ShapeDtypeStruct((G, H), jnp.float32),
        scratch_shapes=[
            pltpu.VMEM((G, H), jnp.float32),
            pltpu.VMEM((1, G), jnp.float32),
            pltpu.VMEM((G, H), jnp.float32),
            pltpu.VMEM((G, 1), jnp.float32),
            pltpu.VMEM((G, 1), jnp.float32),
            pltpu.VMEM((G, H), jnp.float32),
        ],
        compiler_params=pltpu.CompilerParams(
            dimension_semantics=("arbitrary",)),
    )(x, seg,
      W_g1.astype(jnp.bfloat16), bg1, W_g2.astype(jnp.bfloat16),
      W_p[:H], W_p[H:2 * H], W_p[2 * H:], b_p.reshape(1, H),
      gamma.reshape(1, H), beta.reshape(1, H))
    return out
